# baseline (speedup 1.0000x reference)
import jax
import jax.numpy as jnp
from jax.experimental import pallas as pl
from jax.experimental.pallas import tpu as pltpu

_BT = 32


def _mv_body(w_ref, x_ref, o_ref, sw_ref):
    i = pl.program_id(0)

    @pl.when(i == 0)
    def _():
        wv = w_ref[...]
        m = jnp.max(wv)
        e = jnp.exp(wv - m)
        sw_ref[...] = e / jnp.sum(e)

    o_ref[...] = jnp.sum(x_ref[...] * sw_ref[...], axis=1, keepdims=True)


def kernel(x, w, k):
    del k
    t, n = x.shape
    bt = min(_BT, t)
    y = pl.pallas_call(
        _mv_body,
        grid=(t // bt,),
        in_specs=[
            pl.BlockSpec((1, n), lambda i: (0, 0)),
            pl.BlockSpec((bt, n), lambda i: (i, 0)),
        ],
        out_specs=pl.BlockSpec((bt, 1), lambda i: (i, 0)),
        out_shape=jax.ShapeDtypeStruct((t, 1), jnp.float32),
        scratch_shapes=[pltpu.VMEM((1, n), jnp.float32)],
    )(w.reshape(1, n), x)
    return y.reshape(t)


# speedup vs baseline: 1.1440x; 1.1440x over previous
import jax
import jax.numpy as jnp
from jax.experimental import pallas as pl
from jax.experimental.pallas import tpu as pltpu

_BT = 64


def _mv_body(w_ref, x_ref, o_ref, sw_ref):
    i = pl.program_id(0)

    @pl.when(i == 0)
    def _():
        wv = w_ref[...]
        m = jnp.max(wv)
        e = jnp.exp(wv - m)
        sw_ref[...] = e / jnp.sum(e)

    o_ref[...] = jnp.sum(x_ref[...] * sw_ref[...], axis=1, keepdims=True)


def kernel(x, w, k):
    del k
    t, n = x.shape
    bt = min(_BT, t)
    y = pl.pallas_call(
        _mv_body,
        grid=(t // bt,),
        in_specs=[
            pl.BlockSpec((1, n), lambda i: (0, 0)),
            pl.BlockSpec((bt, n), lambda i: (i, 0)),
        ],
        out_specs=pl.BlockSpec((bt, 1), lambda i: (i, 0)),
        out_shape=jax.ShapeDtypeStruct((t, 1), jnp.float32),
        scratch_shapes=[pltpu.VMEM((1, n), jnp.float32)],
    )(w.reshape(1, n), x)
    return y.reshape(t)
